# Initial kernel scaffold; baseline (speedup 1.0000x reference)
import jax
import jax.numpy as jnp
from jax.experimental import pallas as pl


def kernel(tokens, gate_w, w0, b0, w1, b1):
    raise NotImplementedError("write your pallas kernel here")


# speedup vs baseline: 1.0830x; 1.0830x over previous
import functools

import jax
import jax.numpy as jnp
from jax import lax
from jax.experimental import pallas as pl
from jax.experimental.pallas import tpu as pltpu

_NEG_INF = float("-inf")


def _moe_body(x_ref, gw_ref, w0_ref, b0_ref, w1_ref, b1_ref, out_ref):
    x = x_ref[...]
    gw = gw_ref[...]

    logits = lax.dot_general(
        x, gw, (((1,), (1,)), ((), ())),
        preferred_element_type=jnp.float32,
    )
    m = jnp.max(logits, axis=1, keepdims=True)
    e = jnp.exp(logits - m)
    w = e / jnp.sum(e, axis=1, keepdims=True)

    E = w.shape[1]
    col = lax.broadcasted_iota(jnp.int32, w.shape, 1)
    m1 = jnp.max(w, axis=1, keepdims=True)
    i1 = jnp.min(jnp.where(w == m1, col, E), axis=1, keepdims=True)
    w_excl = jnp.where(col == i1, _NEG_INF, w)
    m2 = jnp.max(w_excl, axis=1, keepdims=True)
    i2 = jnp.min(jnp.where((w_excl == m2) & (col != i1), col, E),
                 axis=1, keepdims=True)

    s0 = jnp.where(i1 == 0, w[:, 0:1], 0.0)
    s1 = jnp.where(i2 == 1, w[:, 1:2], 0.0)

    xb = x.astype(jnp.bfloat16)
    h0 = lax.dot_general(
        xb, w0_ref[...], (((1,), (1,)), ((), ())),
        preferred_element_type=jnp.float32,
    ) + b0_ref[...]
    h1 = lax.dot_general(
        xb, w1_ref[...], (((1,), (1,)), ((), ())),
        preferred_element_type=jnp.float32,
    ) + b1_ref[...]
    e0 = h0 * (1.0 / (1.0 + jnp.exp(-h0)))
    e1 = h1 * (1.0 / (1.0 + jnp.exp(-h1)))

    out_ref[...] = s0 * e0 + s1 * e1


@functools.partial(jax.jit, static_argnames=("blkm",))
def _moe_forward(tokens, gate_w, w0, b0, w1, b1, blkm=256):
    batch, seq, hidden = tokens.shape
    m = batch * seq
    x = tokens.reshape(m, hidden)
    nblk = m // blkm
    w0b = w0.astype(jnp.bfloat16)
    w1b = w1.astype(jnp.bfloat16)
    b0r = b0.reshape(1, hidden)
    b1r = b1.reshape(1, hidden)

    out = pl.pallas_call(
        _moe_body,
        grid=(nblk,),
        in_specs=[
            pl.BlockSpec((blkm, hidden), lambda i: (i, 0)),
            pl.BlockSpec((gate_w.shape[0], hidden), lambda i: (0, 0)),
            pl.BlockSpec((hidden, hidden), lambda i: (0, 0)),
            pl.BlockSpec((1, hidden), lambda i: (0, 0)),
            pl.BlockSpec((hidden, hidden), lambda i: (0, 0)),
            pl.BlockSpec((1, hidden), lambda i: (0, 0)),
        ],
        out_specs=pl.BlockSpec((blkm, hidden), lambda i: (i, 0)),
        out_shape=jax.ShapeDtypeStruct((m, hidden), jnp.float32),
        compiler_params=pltpu.CompilerParams(
            dimension_semantics=("arbitrary",),
        ),
    )(x, gate_w, w0b, b0r, w1b, b1r)
    return out.reshape(batch, seq, hidden)


def kernel(tokens, gate_w, w0, b0, w1, b1):
    return _moe_forward(tokens, gate_w, w0, b0, w1, b1)
